# baseline (speedup 1.0000x reference)
import jax
import jax.numpy as jnp
from jax.experimental import pallas as pl
import jax.experimental.pallas.tpu as pltpu

B, H, T = 16, 128, 16000
NBITS = 16
HB = 64


def _kernel(msg_ref, emb_ref, hid_ref, out_ref, aux_ref):
    b = pl.program_id(0)
    hblk = pl.program_id(1)

    @pl.when(jnp.logical_and(b == 0, hblk == 0))
    def _compute_aux():
        msg = msg_ref[...]
        idx = 2 * jax.lax.broadcasted_iota(jnp.int32, (B, NBITS), 1) + msg
        table = jax.lax.broadcasted_iota(jnp.int32, (B, NBITS, 2 * NBITS), 2)
        onehot = (idx[:, :, None] == table).astype(jnp.float32).sum(axis=1)
        aux = jnp.dot(onehot, emb_ref[...],
                      preferred_element_type=jnp.float32)
        eye = (jax.lax.broadcasted_iota(jnp.int32, (H, H), 0)
               == jax.lax.broadcasted_iota(jnp.int32, (H, H), 1))
        masked = aux[:, None, :] * eye[None, :, :].astype(jnp.float32)
        aux_ref[...] = jnp.sum(masked, axis=2, keepdims=True)

    aux_col = aux_ref[b, pl.ds(hblk * HB, HB), :]
    out_ref[...] = hid_ref[...] + aux_col


def kernel(hidden, msg, emb):
    msg = msg.astype(jnp.int32)
    return pl.pallas_call(
        _kernel,
        grid=(B, H // HB),
        in_specs=[
            pl.BlockSpec((B, NBITS), lambda b, h: (0, 0)),
            pl.BlockSpec((2 * NBITS, H), lambda b, h: (0, 0)),
            pl.BlockSpec((None, HB, T), lambda b, h: (b, h, 0)),
        ],
        out_specs=pl.BlockSpec((None, HB, T), lambda b, h: (b, h, 0)),
        out_shape=jax.ShapeDtypeStruct((B, H, T), jnp.float32),
        scratch_shapes=[pltpu.VMEM((B, H, 1), jnp.float32)],
        compiler_params=pltpu.CompilerParams(
            dimension_semantics=("arbitrary", "arbitrary"),
        ),
    )(msg, emb, hidden)


# speedup vs baseline: 1.0147x; 1.0147x over previous
import jax
import jax.numpy as jnp
from jax import lax
from jax.experimental import pallas as pl
import jax.experimental.pallas.tpu as pltpu

B, H, T = 16, 128, 16000
NBITS = 16
CH = 32
CPB = H // CH
TOTAL = B * CPB
NBUF = 3


def _kernel(msg_ref, emb_ref, hid_ref, out_ref,
            in_bufs, out_bufs, aux_ref, in_sems, out_sems):
    msg = msg_ref[...]
    idx = 2 * lax.broadcasted_iota(jnp.int32, (B, NBITS), 1) + msg
    table = lax.broadcasted_iota(jnp.int32, (B, NBITS, 2 * NBITS), 2)
    onehot = (idx[:, :, None] == table).astype(jnp.float32).sum(axis=1)
    aux = jnp.dot(onehot, emb_ref[...],
                  preferred_element_type=jnp.float32)
    eye = (lax.broadcasted_iota(jnp.int32, (H, H), 0)
           == lax.broadcasted_iota(jnp.int32, (H, H), 1))
    masked = aux[:, None, :] * eye[None, :, :].astype(jnp.float32)
    aux_ref[...] = jnp.sum(masked, axis=2, keepdims=True)

    def in_copy(c, slot):
        b = c // CPB
        h0 = (c % CPB) * CH
        return pltpu.make_async_copy(
            hid_ref.at[b, pl.ds(h0, CH), :], in_bufs.at[slot],
            in_sems.at[slot])

    def out_copy(c, slot):
        b = c // CPB
        h0 = (c % CPB) * CH
        return pltpu.make_async_copy(
            out_bufs.at[slot], out_ref.at[b, pl.ds(h0, CH), :],
            out_sems.at[slot])

    for s in range(NBUF):
        in_copy(s, s).start()

    def body(c, carry):
        slot = lax.rem(c, NBUF)
        in_copy(c, slot).wait()

        @pl.when(c >= NBUF)
        def _():
            out_copy(c - NBUF, slot).wait()

        b = c // CPB
        h0 = (c % CPB) * CH
        out_bufs[slot] = in_bufs[slot] + aux_ref[b, pl.ds(h0, CH), :]
        out_copy(c, slot).start()

        @pl.when(c + NBUF < TOTAL)
        def _():
            in_copy(c + NBUF, slot).start()

        return carry

    lax.fori_loop(0, TOTAL, body, 0, unroll=False)

    def drain(c, carry):
        out_copy(c, lax.rem(c, NBUF)).wait()
        return carry

    lax.fori_loop(TOTAL - NBUF, TOTAL, drain, 0, unroll=False)


def kernel(hidden, msg, emb):
    msg = msg.astype(jnp.int32)
    return pl.pallas_call(
        _kernel,
        in_specs=[
            pl.BlockSpec((B, NBITS), lambda: (0, 0)),
            pl.BlockSpec((2 * NBITS, H), lambda: (0, 0)),
            pl.BlockSpec(memory_space=pltpu.MemorySpace.HBM),
        ],
        out_specs=pl.BlockSpec(memory_space=pltpu.MemorySpace.HBM),
        out_shape=jax.ShapeDtypeStruct((B, H, T), jnp.float32),
        scratch_shapes=[
            pltpu.VMEM((NBUF, CH, T), jnp.float32),
            pltpu.VMEM((NBUF, CH, T), jnp.float32),
            pltpu.VMEM((B, H, 1), jnp.float32),
            pltpu.SemaphoreType.DMA((NBUF,)),
            pltpu.SemaphoreType.DMA((NBUF,)),
        ],
    )(msg, emb, hidden)


# speedup vs baseline: 1.0239x; 1.0091x over previous
import jax
import jax.numpy as jnp
from jax import lax
from jax.experimental import pallas as pl
import jax.experimental.pallas.tpu as pltpu

B, H, T = 16, 128, 16000
NBITS = 16
CH = 64
CPB = H // CH
TOTAL = B * CPB
NBUF = 4


def _kernel(msg_ref, emb_ref, hid_ref, out_ref,
            in_bufs, out_bufs, aux_ref, in_sems, out_sems):
    msg = msg_ref[...]
    idx = 2 * lax.broadcasted_iota(jnp.int32, (B, NBITS), 1) + msg
    table = lax.broadcasted_iota(jnp.int32, (B, NBITS, 2 * NBITS), 2)
    onehot = (idx[:, :, None] == table).astype(jnp.float32).sum(axis=1)
    aux = jnp.dot(onehot, emb_ref[...],
                  preferred_element_type=jnp.float32)
    eye = (lax.broadcasted_iota(jnp.int32, (H, H), 0)
           == lax.broadcasted_iota(jnp.int32, (H, H), 1))
    masked = aux[:, None, :] * eye[None, :, :].astype(jnp.float32)
    aux_ref[...] = jnp.sum(masked, axis=2, keepdims=True)

    def in_copy(c, slot):
        b = c // CPB
        h0 = (c % CPB) * CH
        return pltpu.make_async_copy(
            hid_ref.at[b, pl.ds(h0, CH), :], in_bufs.at[slot],
            in_sems.at[slot])

    def out_copy(c, slot):
        b = c // CPB
        h0 = (c % CPB) * CH
        return pltpu.make_async_copy(
            out_bufs.at[slot], out_ref.at[b, pl.ds(h0, CH), :],
            out_sems.at[slot])

    for s in range(NBUF):
        in_copy(s, s).start()

    def body(c, carry):
        slot = lax.rem(c, NBUF)
        in_copy(c, slot).wait()

        @pl.when(c >= NBUF)
        def _():
            out_copy(c - NBUF, slot).wait()

        b = c // CPB
        h0 = (c % CPB) * CH
        out_bufs[slot] = in_bufs[slot] + aux_ref[b, pl.ds(h0, CH), :]
        out_copy(c, slot).start()

        @pl.when(c + NBUF < TOTAL)
        def _():
            in_copy(c + NBUF, slot).start()

        return carry

    lax.fori_loop(0, TOTAL, body, 0, unroll=False)

    def drain(c, carry):
        out_copy(c, lax.rem(c, NBUF)).wait()
        return carry

    lax.fori_loop(TOTAL - NBUF, TOTAL, drain, 0, unroll=False)


def kernel(hidden, msg, emb):
    msg = msg.astype(jnp.int32)
    return pl.pallas_call(
        _kernel,
        in_specs=[
            pl.BlockSpec((B, NBITS), lambda: (0, 0)),
            pl.BlockSpec((2 * NBITS, H), lambda: (0, 0)),
            pl.BlockSpec(memory_space=pltpu.MemorySpace.HBM),
        ],
        out_specs=pl.BlockSpec(memory_space=pltpu.MemorySpace.HBM),
        out_shape=jax.ShapeDtypeStruct((B, H, T), jnp.float32),
        scratch_shapes=[
            pltpu.VMEM((NBUF, CH, T), jnp.float32),
            pltpu.VMEM((NBUF, CH, T), jnp.float32),
            pltpu.VMEM((B, H, 1), jnp.float32),
            pltpu.SemaphoreType.DMA((NBUF,)),
            pltpu.SemaphoreType.DMA((NBUF,)),
        ],
    )(msg, emb, hidden)


# speedup vs baseline: 1.0322x; 1.0081x over previous
import jax
import jax.numpy as jnp
from jax.experimental import pallas as pl
import jax.experimental.pallas.tpu as pltpu

B, H, T = 16, 128, 16000
NBITS = 16


def _kernel(msg_ref, emb_ref, hid_ref, out_ref, aux_ref):
    b = pl.program_id(0)

    @pl.when(b == 0)
    def _compute_aux():
        msg = msg_ref[...]
        idx = 2 * jax.lax.broadcasted_iota(jnp.int32, (B, NBITS), 1) + msg
        table = jax.lax.broadcasted_iota(jnp.int32, (B, NBITS, 2 * NBITS), 2)
        onehot = (idx[:, :, None] == table).astype(jnp.float32).sum(axis=1)
        aux_ref[...] = jnp.dot(onehot, emb_ref[...],
                               preferred_element_type=jnp.float32)

    aux_row = aux_ref[b, :]
    out_ref[...] = hid_ref[...] + aux_row[:, None]


def kernel(hidden, msg, emb):
    msg = msg.astype(jnp.int32)
    return pl.pallas_call(
        _kernel,
        grid=(B,),
        in_specs=[
            pl.BlockSpec((B, NBITS), lambda b: (0, 0)),
            pl.BlockSpec((2 * NBITS, H), lambda b: (0, 0)),
            pl.BlockSpec((None, H, T), lambda b: (b, 0, 0)),
        ],
        out_specs=pl.BlockSpec((None, H, T), lambda b: (b, 0, 0)),
        out_shape=jax.ShapeDtypeStruct((B, H, T), jnp.float32),
        scratch_shapes=[pltpu.VMEM((B, H), jnp.float32)],
        compiler_params=pltpu.CompilerParams(
            dimension_semantics=("arbitrary",),
        ),
    )(msg, emb, hidden)
